# baseline (device time: 1700065 ns/iter reference)
import jax
import jax.numpy as jnp
from jax import lax
from jax.experimental import pallas as pl
from jax.experimental.pallas import tpu as pltpu

N_DEV = 32


def kernel(x, w_mat):
    m, k_per = x.shape
    _, n = w_mat.shape
    ch = m // N_DEV

    def body(x_ref, w_ref, out_ref, xb_ref, wb_ref, send_ref, recv_ref,
             stage_ref, send_sems, recv_sems, copy_sem):
        my = lax.axis_index("i")
        left = (my - 1) % N_DEV
        right = (my + 1) % N_DEV

        barrier_sem = pltpu.get_barrier_semaphore()
        for nbr in (left, right):
            pl.semaphore_signal(barrier_sem, inc=1, device_id=(nbr,),
                                device_id_type=pl.DeviceIdType.MESH)
        pl.semaphore_wait(barrier_sem, 2)

        xb_ref[...] = x_ref[...].astype(jnp.bfloat16)
        wb_ref[...] = w_ref[...].astype(jnp.bfloat16)

        def partial_chunk(c):
            xc = xb_ref[pl.ds(c * ch, ch), :]
            return jnp.dot(xc, wb_ref[...], preferred_element_type=jnp.float32)

        send_ref[0] = partial_chunk(my).astype(jnp.bfloat16)
        for h in range(N_DEV - 1):
            s = h % 2
            r = (h + 1) % 2
            rdma = pltpu.make_async_remote_copy(
                src_ref=send_ref.at[s],
                dst_ref=recv_ref.at[r],
                send_sem=send_sems.at[s],
                recv_sem=recv_sems.at[r],
                device_id=(right,),
                device_id_type=pl.DeviceIdType.MESH,
            )
            rdma.start()
            rdma.wait()
            c = (my - h - 1) % N_DEV
            acc = recv_ref[r].astype(jnp.float32) + partial_chunk(c)
            send_ref[r] = acc.astype(jnp.bfloat16)
            if h == N_DEV - 2:
                stage_ref[...] = acc
                cp = pltpu.make_async_copy(
                    stage_ref, out_ref.at[pl.ds(c * ch, ch), :], copy_sem)
                cp.start()
                cp.wait()

        for g in range(N_DEV - 1):
            rslot = g % 2
            src = send_ref.at[1] if g == 0 else recv_ref.at[(g - 1) % 2]
            rdma = pltpu.make_async_remote_copy(
                src_ref=src,
                dst_ref=recv_ref.at[rslot],
                send_sem=send_sems.at[g % 2],
                recv_sem=recv_sems.at[rslot],
                device_id=(right,),
                device_id_type=pl.DeviceIdType.MESH,
            )
            rdma.start()
            rdma.wait()
            c = (my - g) % N_DEV
            stage_ref[...] = recv_ref[rslot].astype(jnp.float32)
            cp = pltpu.make_async_copy(
                stage_ref, out_ref.at[pl.ds(c * ch, ch), :], copy_sem)
            cp.start()
            cp.wait()

    return pl.pallas_call(
        body,
        out_shape=jax.ShapeDtypeStruct((m, n), jnp.float32),
        in_specs=[
            pl.BlockSpec(memory_space=pltpu.MemorySpace.VMEM),
            pl.BlockSpec(memory_space=pltpu.MemorySpace.VMEM),
        ],
        out_specs=pl.BlockSpec(memory_space=pltpu.MemorySpace.HBM),
        scratch_shapes=[
            pltpu.VMEM((m, k_per), jnp.bfloat16),
            pltpu.VMEM((k_per, n), jnp.bfloat16),
            pltpu.VMEM((2, ch, n), jnp.bfloat16),
            pltpu.VMEM((2, ch, n), jnp.bfloat16),
            pltpu.VMEM((ch, n), jnp.float32),
            pltpu.SemaphoreType.DMA((2,)),
            pltpu.SemaphoreType.DMA((2,)),
            pltpu.SemaphoreType.DMA,
        ],
        compiler_params=pltpu.CompilerParams(collective_id=0),
    )(x, w_mat)


# device time: 1626188 ns/iter; 1.0454x vs baseline; 1.0454x over previous
import jax
import jax.numpy as jnp
from jax import lax
from jax.experimental import pallas as pl
from jax.experimental.pallas import tpu as pltpu

N_DEV = 32
F32 = jnp.float32
BF16 = jnp.bfloat16


def kernel(x, w_mat):
    m, k_per = x.shape
    _, n = w_mat.shape
    ch = m // N_DEV
    hh = ch // 2

    def body(x_ref, w_ref, out_ref, xb_ref, wb_ref,
             send_r, recv_r, send_l, recv_l, stage_r, stage_l,
             sems_sr, sems_rr, sems_sl, sems_rl, cp_sems_r, cp_sems_l,
             credit_r, credit_l):
        my = lax.axis_index("i")
        left = (my - 1) % N_DEV
        right = (my + 1) % N_DEV

        barrier_sem = pltpu.get_barrier_semaphore()
        for nbr in (left, right):
            pl.semaphore_signal(barrier_sem, inc=1, device_id=(nbr,),
                                device_id_type=pl.DeviceIdType.MESH)
        pl.semaphore_wait(barrier_sem, 2)

        xb_ref[...] = x_ref[...].astype(BF16)
        wb_ref[...] = w_ref[...].astype(BF16)

        def partial_half(c, top):
            off = c * ch + (0 if top else hh)
            return jnp.dot(xb_ref[pl.ds(off, hh), :], wb_ref[...],
                           preferred_element_type=F32)

        pend_r = [None, None]
        pend_l = [None, None]

        def emit_out(slot, acc_r_f32, acc_l_f32, cr, cl):
            for pend, stage, sems, acc, row0 in (
                (pend_r, stage_r, cp_sems_r, acc_r_f32, cr * ch),
                (pend_l, stage_l, cp_sems_l, acc_l_f32, cl * ch + hh),
            ):
                if pend[slot] is not None:
                    pend[slot].wait()
                stage[slot] = acc
                cp = pltpu.make_async_copy(
                    stage.at[slot], out_ref.at[pl.ds(row0, hh), :],
                    sems.at[slot])
                cp.start()
                pend[slot] = cp

        def grant_credits():
            pl.semaphore_signal(credit_r, inc=1, device_id=(left,),
                                device_id_type=pl.DeviceIdType.MESH)
            pl.semaphore_signal(credit_l, inc=1, device_id=(right,),
                                device_id_type=pl.DeviceIdType.MESH)

        def take_credits():
            pl.semaphore_wait(credit_r, 1)
            pl.semaphore_wait(credit_l, 1)

        send_r[0] = partial_half(my, True).astype(BF16)
        send_l[0] = partial_half(my, False).astype(BF16)
        for h in range(N_DEV - 1):
            s = h % 2
            r = (h + 1) % 2
            rd_r = pltpu.make_async_remote_copy(
                src_ref=send_r.at[s], dst_ref=recv_r.at[r],
                send_sem=sems_sr.at[s], recv_sem=sems_rr.at[r],
                device_id=(right,), device_id_type=pl.DeviceIdType.MESH)
            rd_l = pltpu.make_async_remote_copy(
                src_ref=send_l.at[s], dst_ref=recv_l.at[r],
                send_sem=sems_sl.at[s], recv_sem=sems_rl.at[r],
                device_id=(left,), device_id_type=pl.DeviceIdType.MESH)
            if h >= 2:
                take_credits()
            rd_r.start()
            rd_l.start()
            cr = (my - h - 1) % N_DEV
            cl = (my + h + 1) % N_DEV
            pr = partial_half(cr, True)
            pf = partial_half(cl, False)
            rd_r.wait()
            rd_l.wait()
            acc_r = recv_r[r].astype(F32) + pr
            acc_l = recv_l[r].astype(F32) + pf
            send_r[r] = acc_r.astype(BF16)
            send_l[r] = acc_l.astype(BF16)
            grant_credits()
            if h == N_DEV - 2:
                emit_out(0, acc_r, acc_l, cr, cl)

        for g in range(N_DEV - 1):
            rslot = g % 2
            src_r = send_r.at[1] if g == 0 else recv_r.at[(g - 1) % 2]
            src_l = send_l.at[1] if g == 0 else recv_l.at[(g - 1) % 2]
            rd_r = pltpu.make_async_remote_copy(
                src_ref=src_r, dst_ref=recv_r.at[rslot],
                send_sem=sems_sr.at[g % 2], recv_sem=sems_rr.at[rslot],
                device_id=(right,), device_id_type=pl.DeviceIdType.MESH)
            rd_l = pltpu.make_async_remote_copy(
                src_ref=src_l, dst_ref=recv_l.at[rslot],
                send_sem=sems_sl.at[g % 2], recv_sem=sems_rl.at[rslot],
                device_id=(left,), device_id_type=pl.DeviceIdType.MESH)
            take_credits()
            rd_r.start()
            rd_l.start()
            rd_r.wait()
            rd_l.wait()
            if 1 <= g <= N_DEV - 3:
                grant_credits()
            cr = (my - g) % N_DEV
            cl = (my + g) % N_DEV
            emit_out((g + 1) % 2,
                     recv_r[rslot].astype(F32), recv_l[rslot].astype(F32),
                     cr, cl)

        for pend in (pend_r, pend_l):
            for cp in pend:
                if cp is not None:
                    cp.wait()

    return pl.pallas_call(
        body,
        out_shape=jax.ShapeDtypeStruct((m, n), F32),
        in_specs=[
            pl.BlockSpec(memory_space=pltpu.MemorySpace.VMEM),
            pl.BlockSpec(memory_space=pltpu.MemorySpace.VMEM),
        ],
        out_specs=pl.BlockSpec(memory_space=pltpu.MemorySpace.HBM),
        scratch_shapes=[
            pltpu.VMEM((m, k_per), BF16),
            pltpu.VMEM((k_per, n), BF16),
            pltpu.VMEM((2, hh, n), BF16),
            pltpu.VMEM((2, hh, n), BF16),
            pltpu.VMEM((2, hh, n), BF16),
            pltpu.VMEM((2, hh, n), BF16),
            pltpu.VMEM((2, hh, n), F32),
            pltpu.VMEM((2, hh, n), F32),
            pltpu.SemaphoreType.DMA((2,)),
            pltpu.SemaphoreType.DMA((2,)),
            pltpu.SemaphoreType.DMA((2,)),
            pltpu.SemaphoreType.DMA((2,)),
            pltpu.SemaphoreType.DMA((2,)),
            pltpu.SemaphoreType.DMA((2,)),
            pltpu.SemaphoreType.REGULAR,
            pltpu.SemaphoreType.REGULAR,
        ],
        compiler_params=pltpu.CompilerParams(collective_id=0),
    )(x, w_mat)
